# baseline (speedup 1.0000x reference)
import jax
import jax.numpy as jnp
from jax import lax
from jax.experimental import pallas as pl
from jax.experimental.pallas import tpu as pltpu
from jax.experimental.pallas import tpu_sc as plsc

IVS = 100000
POOL_W = 2048
NKEY = 64 * POOL_W
T = 16384
H = 128
U = 512
S = 8
NC, NS, L = 2, 16, 16
NW = NC * NS
TPW = T // NW
CH = 256
HB = 64
UPW = U // NS


def _body(ids_hbm, embed_hbm, buf_hbm, upd_hbm, ui_hbm, utb_hbm, hwpi_hbm,
          tb_hbm, out_hbm,
          hwpi_v, uidx_v, utb_v, ui_v, keys_v, enc_v,
          ids0_v, ids1_v, tb0_v, tb1_v,
          eidx0_v, eidx1_v, hkey0_v, hkey1_v, hpos0_v, hpos1_v,
          mapw_v, base_a, base_b, hyp0_v, hyp1_v, srow_v,
          enc_sh, sem_e, sem_g0, sem_g1, sem_w, sem_o):
  cid = lax.axis_index("c")
  sid = lax.axis_index("s")
  wid = sid * NC + cid
  iota = lax.iota(jnp.int32, L)
  neg1 = jnp.full((L,), -1, jnp.int32)
  tbase = wid * TPW

  scope = jax.named_scope
  with scope("p_stage"):
    pltpu.sync_copy(hwpi_hbm, hwpi_v)
    for r in range(2):
      pltpu.sync_copy(upd_hbm.at[pl.ds(sid * UPW * S + r * 128, 128)],
                      uidx_v.at[r])
    pltpu.sync_copy(utb_hbm, utb_v)
    pltpu.sync_copy(ui_hbm, ui_v)
    pltpu.sync_copy(ids_hbm.at[pl.ds(tbase, CH)], ids0_v)
    pltpu.sync_copy(ids_hbm.at[pl.ds(tbase + CH, CH)], ids1_v)
    pltpu.sync_copy(tb_hbm.at[pl.ds(tbase, CH)], tb0_v)
    pltpu.sync_copy(tb_hbm.at[pl.ds(tbase + CH, CH)], tb1_v)

    enc_cps = [pltpu.async_copy(embed_hbm.at[uidx_v.at[r]],
                                base_a.at[pl.ds(r * 128, 128), :], sem_e)
               for r in range(2)]

  def pass1(ids_v, tbv_v, eidx_v, hkey_v, hpos_v):
    for r in range(4):
      for j in range(4):
        hkey_v[r, pl.ds(j * L, L)] = iota + r * HB + j * L
    nh = jnp.int32(0)
    for g in range(CH // L):
      ids = ids_v[pl.ds(g * L, L)]
      tb = tbv_v[pl.ds(g * L, L)]
      slot = plsc.load_gather(hwpi_v, [tb])
      ishyp = ids >= IVS
      key = slot * POOL_W + (ids - IVS)
      incl = plsc.cumsum(jnp.where(ishyp, 1, 0))
      rank = nh + incl - 1
      rclip = jnp.clip(rank, 0, CH - 1)
      plsc.store_scatter(hkey_v, [rclip >> 6, rclip & (HB - 1)], key,
                         mask=ishyp)
      plsc.store_scatter(hpos_v, [rclip], g * L + iota, mask=ishyp)
      eidx_v[g // 8, pl.ds((g % 8) * L, L)] = jnp.where(
          ishyp, g * L + iota, ids)
      nh = nh + jnp.sum(jnp.where(ishyp, 1, 0))
    return nh

  with scope("p_pass1"):
    nh0 = pass1(ids0_v, tb0_v, eidx0_v, hkey0_v, hpos0_v)
    nh1 = pass1(ids1_v, tb1_v, eidx1_v, hkey1_v, hpos1_v)

  with scope("p_encwait"):
    for cp in enc_cps:
      cp.wait()
  with scope("p_encode"):
    for g in range(UPW // L):
      subcol = []
      submask = []
      for ss in range(S):
        col = iota * S + ss
        iv = plsc.load_gather(uidx_v, [jnp.full((L,), g, jnp.int32), col])
        subcol.append(col)
        submask.append(iv != 0)
      cnt = jnp.zeros((L,), jnp.float32)
      for ss in range(S):
        cnt = cnt + jnp.where(submask[ss], 1.0, 0.0)
      recip = 1.0 / jnp.maximum(cnt, 1.0)

      def enc_col(col, _):
        colv = jnp.full((L,), col, jnp.int32)
        acc = jnp.zeros((L,), jnp.float32)
        for ss in range(S):
          v = plsc.load_gather(base_a, [g * 128 + subcol[ss], colv])
          acc = acc + jnp.where(submask[ss], v, 0.0)
        plsc.store_scatter(enc_v, [g * L + iota, colv], acc * recip)
        return _

      lax.fori_loop(0, H, enc_col, 0)
    pltpu.sync_copy(enc_v, enc_sh.at[pl.ds(sid * UPW, UPW), :])

  cps0 = [pltpu.async_copy(embed_hbm.at[eidx0_v.at[r]],
                           base_a.at[pl.ds(r * 128, 128), :], sem_g0)
          for r in range(2)]
  cps0.append(pltpu.async_copy(buf_hbm.at[hkey0_v.at[0]], hyp0_v, sem_g0))
  cps1 = [pltpu.async_copy(embed_hbm.at[eidx1_v.at[r]],
                           base_b.at[pl.ds(r * 128, 128), :], sem_g1)
          for r in range(2)]
  cps1.append(pltpu.async_copy(buf_hbm.at[hkey1_v.at[0]], hyp1_v, sem_g1))

  with scope("p_keys"):
    for g in range(U // L):
      utb_g = utb_v[pl.ds(g * L, L)]
      slot = plsc.load_gather(hwpi_v, [utb_g])
      keys_v[pl.ds(g * L, L)] = slot * POOL_W + ui_v[pl.ds(g * L, L)]

  with scope("p_barrier"):
    plsc.subcore_barrier()

  def compute(base_v, hyp_v, hkey_v, hpos_v, nh, cps):
    with scope("p_gwait"):
      for cp in cps:
        cp.wait()
    for b in range(CH // HB):
      if b > 0:
        @pl.when(b * HB < nh)
        def _fetch_batch():
          pltpu.async_copy(buf_hbm.at[hkey_v.at[b]], hyp_v, sem_o).wait()
      for gg in range(HB // L):
        bb = b * (HB // L) + gg

        @pl.when(bb * L < nh)
        def _group():
          hk = hkey_v[b, pl.ds(gg * L, L)]
          lanev = bb * L + iota < nh

          def find_body(i, u):
            for d in range(8):
              p = i * 8 + d
              pk = plsc.load_gather(keys_v, [jnp.full((L,), p, jnp.int32)])
              u = jnp.where(pk == hk, p, u)
            return u

          u = lax.fori_loop(0, U // 8, find_body, neg1)
          valid = jnp.logical_and(u >= 0, lanev)

          @pl.when(jnp.any(valid))
          def _override():
            mapw_v[...] = jnp.clip(u, 0, U - 1)
            pltpu.async_copy(enc_sh.at[mapw_v], srow_v, sem_o).wait()

            def ov_col(c4, _):
              for d in range(4):
                colv = jnp.full((L,), c4 * 4 + d, jnp.int32)
                v = plsc.load_gather(srow_v, [iota, colv], mask=valid)
                plsc.store_scatter(hyp_v, [gg * L + iota, colv], v,
                                   mask=valid)
              return _

            lax.fori_loop(0, H // 4, ov_col, 0)

          pos = jnp.clip(hpos_v[pl.ds(bb * L, L)], 0, CH - 1)

          def rp_col(c4, _):
            for d in range(4):
              colv = jnp.full((L,), c4 * 4 + d, jnp.int32)
              v = plsc.load_gather(hyp_v, [gg * L + iota, colv], mask=lanev)
              plsc.store_scatter(base_v, [pos, colv], v, mask=lanev)
            return _

          lax.fori_loop(0, H // 4, rp_col, 0)

  with scope("p_chunk0"):
    compute(base_a, hyp0_v, hkey0_v, hpos0_v, nh0, cps0)
    w0 = pltpu.async_copy(base_a, out_hbm.at[pl.ds(tbase, CH), :], sem_w)
  with scope("p_chunk1"):
    compute(base_b, hyp1_v, hkey1_v, hpos1_v, nh1, cps1)
  with scope("p_tail"):
    w0.wait()
    pltpu.sync_copy(base_b, out_hbm.at[pl.ds(tbase + CH, CH), :])


@jax.jit
def _run(ids, embed, buf2d, upd_flat, ui, utb, hwpi, tb):
  mesh = plsc.VectorSubcoreMesh(core_axis_name="c", subcore_axis_name="s",
                                num_cores=NC, num_subcores=NS)
  scratch = [
      pltpu.VMEM((64,), jnp.int32),
      pltpu.VMEM((2, 128), jnp.int32),
      pltpu.VMEM((U,), jnp.int32),
      pltpu.VMEM((U,), jnp.int32),
      pltpu.VMEM((U,), jnp.int32),
      pltpu.VMEM((UPW, H), jnp.float32),
      pltpu.VMEM((CH,), jnp.int32),
      pltpu.VMEM((CH,), jnp.int32),
      pltpu.VMEM((CH,), jnp.int32),
      pltpu.VMEM((CH,), jnp.int32),
      pltpu.VMEM((2, 128), jnp.int32),
      pltpu.VMEM((2, 128), jnp.int32),
      pltpu.VMEM((4, HB), jnp.int32),
      pltpu.VMEM((4, HB), jnp.int32),
      pltpu.VMEM((CH,), jnp.int32),
      pltpu.VMEM((CH,), jnp.int32),
      pltpu.VMEM((L,), jnp.int32),
      pltpu.VMEM((CH, H), jnp.float32),
      pltpu.VMEM((CH, H), jnp.float32),
      pltpu.VMEM((HB, H), jnp.float32),
      pltpu.VMEM((HB, H), jnp.float32),
      pltpu.VMEM((L, H), jnp.float32),
      pltpu.VMEM_SHARED((U, H), jnp.float32),
      pltpu.SemaphoreType.DMA,
      pltpu.SemaphoreType.DMA,
      pltpu.SemaphoreType.DMA,
      pltpu.SemaphoreType.DMA,
      pltpu.SemaphoreType.DMA,
  ]
  f = pl.kernel(
      _body,
      out_type=jax.ShapeDtypeStruct((T, H), jnp.float32),
      mesh=mesh,
      scratch_types=scratch,
      compiler_params=pltpu.CompilerParams(needs_layout_passes=False),
  )
  return f(ids, embed, buf2d, upd_flat, ui, utb, hwpi, tb)


def kernel(input_, embed_weight, embedding_buffer, updates, updates_indices,
           update_to_batch, hyper_weight_pool_indices, token_to_batch_indices):
  ids = input_.astype(jnp.int32)
  buf2d = embedding_buffer.reshape(NKEY, H)
  upd_flat = updates.astype(jnp.int32).reshape(U * S)
  return _run(ids, embed_weight, buf2d, upd_flat,
              updates_indices.astype(jnp.int32),
              update_to_batch.astype(jnp.int32),
              hyper_weight_pool_indices.astype(jnp.int32),
              token_to_batch_indices.astype(jnp.int32))


# speedup vs baseline: 1.1036x; 1.1036x over previous
import jax
import jax.numpy as jnp
from jax import lax
from jax.experimental import pallas as pl
from jax.experimental.pallas import tpu as pltpu
from jax.experimental.pallas import tpu_sc as plsc

IVS = 100000
POOL_W = 2048
NKEY = 64 * POOL_W
T = 16384
H = 128
U = 512
S = 8
NC, NS, L = 2, 16, 16
NW = NC * NS
TPW = T // NW
CH = 256
HB = 64
UPW = U // NS


def _body(ids_hbm, embed_hbm, buf_hbm, upd_hbm, ui_hbm, utb_hbm, hwpi_hbm,
          tb_hbm, out_hbm,
          hwpi_v, uidx_v, utb_v, ui_v, keys_v, enc_v,
          ids0_v, ids1_v, tb0_v, tb1_v,
          eidx0_v, eidx1_v, hkey0_v, hkey1_v, hpos0_v, hpos1_v,
          mapw_v, w0_v, base_a, base_b, hyp0_v, hyp1_v, srow_v,
          enc_sh, sem_e, sem_g0, sem_g1, sem_w, sem_o):
  cid = lax.axis_index("c")
  sid = lax.axis_index("s")
  wid = sid * NC + cid
  iota = lax.iota(jnp.int32, L)
  neg1 = jnp.full((L,), -1, jnp.int32)
  tbase = wid * TPW

  scope = jax.named_scope
  with scope("p_stage"):
    ucps = [pltpu.async_copy(upd_hbm.at[pl.ds(sid * UPW * S + r * 128, 128)],
                             uidx_v.at[r], sem_e) for r in range(2)]
    stage_cps = [
        pltpu.async_copy(hwpi_hbm, hwpi_v, sem_o),
        pltpu.async_copy(utb_hbm, utb_v, sem_o),
        pltpu.async_copy(ui_hbm, ui_v, sem_o),
        pltpu.async_copy(ids_hbm.at[pl.ds(tbase, CH)], ids0_v, sem_o),
        pltpu.async_copy(ids_hbm.at[pl.ds(tbase + CH, CH)], ids1_v, sem_o),
        pltpu.async_copy(tb_hbm.at[pl.ds(tbase, CH)], tb0_v, sem_o),
        pltpu.async_copy(tb_hbm.at[pl.ds(tbase + CH, CH)], tb1_v, sem_o),
        pltpu.async_copy(embed_hbm.at[pl.ds(0, 1), :], w0_v, sem_o),
    ]
    for cp in ucps:
      cp.wait()
    enc_cps = [pltpu.async_copy(embed_hbm.at[uidx_v.at[r]],
                                base_a.at[pl.ds(r * 128, 128), :], sem_e)
               for r in range(2)]
    for cp in stage_cps:
      cp.wait()

  def pass1(ids_v, tbv_v, eidx_v, hkey_v, hpos_v):
    for r in range(4):
      for j in range(4):
        hkey_v[r, pl.ds(j * L, L)] = iota + r * HB + j * L
    nh = jnp.int32(0)
    for g in range(CH // L):
      ids = ids_v[pl.ds(g * L, L)]
      tb = tbv_v[pl.ds(g * L, L)]
      slot = plsc.load_gather(hwpi_v, [tb])
      ishyp = ids >= IVS
      key = slot * POOL_W + (ids - IVS)
      incl = plsc.cumsum(jnp.where(ishyp, 1, 0))
      rank = nh + incl - 1
      rclip = jnp.clip(rank, 0, CH - 1)
      plsc.store_scatter(hkey_v, [rclip >> 6, rclip & (HB - 1)], key,
                         mask=ishyp)
      plsc.store_scatter(hpos_v, [rclip], g * L + iota, mask=ishyp)
      eidx_v[g // 8, pl.ds((g % 8) * L, L)] = jnp.where(
          ishyp, g * L + iota, ids)
      nh = nh + jnp.sum(jnp.where(ishyp, 1, 0))
    return nh

  with scope("p_pass1"):
    nh0 = pass1(ids0_v, tb0_v, eidx0_v, hkey0_v, hpos0_v)
    nh1 = pass1(ids1_v, tb1_v, eidx1_v, hkey1_v, hpos1_v)

  with scope("p_encwait"):
    for cp in enc_cps:
      cp.wait()
  with scope("p_encode"):
    zero16 = jnp.zeros((L,), jnp.int32)
    for g in range(UPW // L):
      cnt = jnp.zeros((L,), jnp.float32)
      rowvs = []
      for ss in range(S):
        col = iota * S + ss
        iv = plsc.load_gather(uidx_v, [jnp.full((L,), g, jnp.int32), col])
        rowvs.append(g * 128 + col)
        cnt = cnt + jnp.where(iv != 0, 1.0, 0.0)
      npadv = float(S) - cnt
      recip = 1.0 / jnp.maximum(cnt, 1.0)
      encrow = g * L + iota

      def enc_col(c4, _):
        for d in range(4):
          colv = jnp.full((L,), c4 * 4 + d, jnp.int32)
          acc = plsc.load_gather(base_a, [rowvs[0], colv])
          for ss in range(1, S):
            acc = acc + plsc.load_gather(base_a, [rowvs[ss], colv])
          w0c = plsc.load_gather(w0_v, [zero16, colv])
          plsc.store_scatter(enc_v, [encrow, colv],
                             (acc - npadv * w0c) * recip)
        return _

      lax.fori_loop(0, H // 4, enc_col, 0)
    pltpu.sync_copy(enc_v, enc_sh.at[pl.ds(sid * UPW, UPW), :])

  cps0 = [pltpu.async_copy(embed_hbm.at[eidx0_v.at[r]],
                           base_a.at[pl.ds(r * 128, 128), :], sem_g0)
          for r in range(2)]
  cps0.append(pltpu.async_copy(buf_hbm.at[hkey0_v.at[0]], hyp0_v, sem_g0))
  cps1 = [pltpu.async_copy(embed_hbm.at[eidx1_v.at[r]],
                           base_b.at[pl.ds(r * 128, 128), :], sem_g1)
          for r in range(2)]
  cps1.append(pltpu.async_copy(buf_hbm.at[hkey1_v.at[0]], hyp1_v, sem_g1))

  with scope("p_keys"):
    for g in range(U // L):
      utb_g = utb_v[pl.ds(g * L, L)]
      slot = plsc.load_gather(hwpi_v, [utb_g])
      keys_v[pl.ds(g * L, L)] = slot * POOL_W + ui_v[pl.ds(g * L, L)]

  with scope("p_barrier"):
    plsc.subcore_barrier()

  NG = HB // L

  def compute(base_v, hyp_v, hkey_v, hpos_v, nh, cps):
    with scope("p_gwait"):
      for cp in cps:
        cp.wait()
    for b in range(CH // HB):

      @pl.when(b * HB < nh)
      def _batch():
        if b > 0:
          pltpu.async_copy(buf_hbm.at[hkey_v.at[b]], hyp_v, sem_o).wait()
        hks = [hkey_v[b, pl.ds(j * L, L)] for j in range(NG)]
        lanevs = [b * HB + j * L + iota < nh for j in range(NG)]

        def find_body(i, us):
          for d in range(8):
            p = i * 8 + d
            pk = plsc.load_gather(keys_v, [jnp.full((L,), p, jnp.int32)])
            us = tuple(jnp.where(pk == hks[j], p, us[j]) for j in range(NG))
          return us

        us = lax.fori_loop(0, U // 8, find_body, (neg1,) * NG)
        valids = [jnp.logical_and(us[j] >= 0, lanevs[j]) for j in range(NG)]
        anyv = jnp.int32(0)
        for j in range(NG):
          anyv = anyv + jnp.sum(jnp.where(valids[j], 1, 0))

        @pl.when(anyv > 0)
        def _override():
          for j in range(NG):
            mapw_v[pl.ds(j * L, L)] = jnp.clip(us[j], 0, U - 1)
          pltpu.async_copy(enc_sh.at[mapw_v], srow_v, sem_o).wait()

          def ov_col(col, _):
            colv = jnp.full((L,), col, jnp.int32)
            for j in range(NG):
              v = plsc.load_gather(srow_v, [j * L + iota, colv],
                                   mask=valids[j])
              plsc.store_scatter(hyp_v, [j * L + iota, colv], v,
                                 mask=valids[j])
            return _

          lax.fori_loop(0, H, ov_col, 0)

        poss = [jnp.clip(hpos_v[pl.ds(b * HB + j * L, L)], 0, CH - 1)
                for j in range(NG)]

        def rp_col(col, _):
          colv = jnp.full((L,), col, jnp.int32)
          for j in range(NG):
            v = plsc.load_gather(hyp_v, [j * L + iota, colv],
                                 mask=lanevs[j])
            plsc.store_scatter(base_v, [poss[j], colv], v, mask=lanevs[j])
          return _

        lax.fori_loop(0, H, rp_col, 0)

  with scope("p_chunk0"):
    compute(base_a, hyp0_v, hkey0_v, hpos0_v, nh0, cps0)
    w0 = pltpu.async_copy(base_a, out_hbm.at[pl.ds(tbase, CH), :], sem_w)
  with scope("p_chunk1"):
    compute(base_b, hyp1_v, hkey1_v, hpos1_v, nh1, cps1)
  with scope("p_tail"):
    w0.wait()
    pltpu.sync_copy(base_b, out_hbm.at[pl.ds(tbase + CH, CH), :])


@jax.jit
def _run(ids, embed, buf2d, upd_flat, ui, utb, hwpi, tb):
  mesh = plsc.VectorSubcoreMesh(core_axis_name="c", subcore_axis_name="s",
                                num_cores=NC, num_subcores=NS)
  scratch = [
      pltpu.VMEM((64,), jnp.int32),
      pltpu.VMEM((2, 128), jnp.int32),
      pltpu.VMEM((U,), jnp.int32),
      pltpu.VMEM((U,), jnp.int32),
      pltpu.VMEM((U,), jnp.int32),
      pltpu.VMEM((UPW, H), jnp.float32),
      pltpu.VMEM((CH,), jnp.int32),
      pltpu.VMEM((CH,), jnp.int32),
      pltpu.VMEM((CH,), jnp.int32),
      pltpu.VMEM((CH,), jnp.int32),
      pltpu.VMEM((2, 128), jnp.int32),
      pltpu.VMEM((2, 128), jnp.int32),
      pltpu.VMEM((4, HB), jnp.int32),
      pltpu.VMEM((4, HB), jnp.int32),
      pltpu.VMEM((CH,), jnp.int32),
      pltpu.VMEM((CH,), jnp.int32),
      pltpu.VMEM((HB,), jnp.int32),
      pltpu.VMEM((1, H), jnp.float32),
      pltpu.VMEM((CH, H), jnp.float32),
      pltpu.VMEM((CH, H), jnp.float32),
      pltpu.VMEM((HB, H), jnp.float32),
      pltpu.VMEM((HB, H), jnp.float32),
      pltpu.VMEM((HB, H), jnp.float32),
      pltpu.VMEM_SHARED((U, H), jnp.float32),
      pltpu.SemaphoreType.DMA,
      pltpu.SemaphoreType.DMA,
      pltpu.SemaphoreType.DMA,
      pltpu.SemaphoreType.DMA,
      pltpu.SemaphoreType.DMA,
  ]
  f = pl.kernel(
      _body,
      out_type=jax.ShapeDtypeStruct((T, H), jnp.float32),
      mesh=mesh,
      scratch_types=scratch,
      compiler_params=pltpu.CompilerParams(needs_layout_passes=False),
  )
  return f(ids, embed, buf2d, upd_flat, ui, utb, hwpi, tb)


def kernel(input_, embed_weight, embedding_buffer, updates, updates_indices,
           update_to_batch, hyper_weight_pool_indices, token_to_batch_indices):
  ids = input_.astype(jnp.int32)
  buf2d = embedding_buffer.reshape(NKEY, H)
  upd_flat = updates.astype(jnp.int32).reshape(U * S)
  return _run(ids, embed_weight, buf2d, upd_flat,
              updates_indices.astype(jnp.int32),
              update_to_batch.astype(jnp.int32),
              hyper_weight_pool_indices.astype(jnp.int32),
              token_to_batch_indices.astype(jnp.int32))


# speedup vs baseline: 1.5776x; 1.4296x over previous
import jax
import jax.numpy as jnp
from jax import lax
from jax.experimental import pallas as pl
from jax.experimental.pallas import tpu as pltpu
from jax.experimental.pallas import tpu_sc as plsc

IVS = 100000
POOL_W = 2048
NKEY = 64 * POOL_W
T = 16384
H = 128
U = 512
S = 8
NC, NS, L = 2, 16, 16
NW = NC * NS
TPW = T // NW
CH = 256
HB = 64
UPW = U // NS


def _body(ids_hbm, embed_hbm, buf_hbm, upd_hbm, ui_hbm, utb_hbm, hwpi_hbm,
          tb_hbm, out_hbm,
          hwpi_v, uidx_v, utb_v, ui_v, keys_v, enc_v,
          ids0_v, ids1_v, tb0_v, tb1_v,
          eidx0_v, eidx1_v, hkey0_v, hkey1_v, hpos0_v, hpos1_v,
          mapw_v, uval_v, npad_v, recip_v, w0_v,
          base_a, base_b, hyp0_v, hyp1_v, srow_v,
          enc_sh, sem_e, sem_g0, sem_g1, sem_w, sem_o):
  cid = lax.axis_index("c")
  sid = lax.axis_index("s")
  wid = sid * NC + cid
  iota = lax.iota(jnp.int32, L)
  neg1 = jnp.full((L,), -1, jnp.int32)
  tbase = wid * TPW

  scope = jax.named_scope
  with scope("p_stage"):
    ucps = [pltpu.async_copy(upd_hbm.at[pl.ds(sid * UPW * S + r * 128, 128)],
                             uidx_v.at[r], sem_e) for r in range(2)]
    stage_cps = [
        pltpu.async_copy(hwpi_hbm, hwpi_v, sem_o),
        pltpu.async_copy(utb_hbm, utb_v, sem_o),
        pltpu.async_copy(ui_hbm, ui_v, sem_o),
        pltpu.async_copy(ids_hbm.at[pl.ds(tbase, CH)], ids0_v, sem_o),
        pltpu.async_copy(ids_hbm.at[pl.ds(tbase + CH, CH)], ids1_v, sem_o),
        pltpu.async_copy(tb_hbm.at[pl.ds(tbase, CH)], tb0_v, sem_o),
        pltpu.async_copy(tb_hbm.at[pl.ds(tbase + CH, CH)], tb1_v, sem_o),
        pltpu.async_copy(embed_hbm.at[pl.ds(0, 1), :], w0_v, sem_o),
    ]
    for cp in ucps:
      cp.wait()
    enc_cps = [pltpu.async_copy(embed_hbm.at[uidx_v.at[r]],
                                base_a.at[pl.ds(r * 128, 128), :], sem_e)
               for r in range(2)]
    for cp in stage_cps:
      cp.wait()

  def pass1(ids_v, tbv_v, eidx_v, hkey_v, hpos_v):
    for r in range(4):
      for j in range(4):
        hkey_v[r, pl.ds(j * L, L)] = iota + r * HB + j * L
    nh = jnp.int32(0)
    for g in range(CH // L):
      ids = ids_v[pl.ds(g * L, L)]
      tb = tbv_v[pl.ds(g * L, L)]
      slot = plsc.load_gather(hwpi_v, [tb])
      ishyp = ids >= IVS
      key = slot * POOL_W + (ids - IVS)
      incl = plsc.cumsum(jnp.where(ishyp, 1, 0))
      rank = nh + incl - 1
      rclip = jnp.clip(rank, 0, CH - 1)
      plsc.store_scatter(hkey_v, [rclip >> 6, rclip & (HB - 1)], key,
                         mask=ishyp)
      plsc.store_scatter(hpos_v, [rclip], g * L + iota, mask=ishyp)
      eidx_v[g // 8, pl.ds((g % 8) * L, L)] = jnp.where(
          ishyp, g * L + iota, ids)
      nh = nh + jnp.sum(jnp.where(ishyp, 1, 0))
    return nh

  with scope("p_pass1"):
    nh0 = pass1(ids0_v, tb0_v, eidx0_v, hkey0_v, hpos0_v)
    nh1 = pass1(ids1_v, tb1_v, eidx1_v, hkey1_v, hpos1_v)

  with scope("p_encwait"):
    for cp in enc_cps:
      cp.wait()
  with scope("p_encode"):
    zero16 = jnp.zeros((L,), jnp.int32)
    colvs = [c * L + iota for c in range(H // L)]
    for g in range(UPW // L):
      cnt = jnp.zeros((L,), jnp.float32)
      for ss in range(S):
        iv = plsc.load_gather(uidx_v, [jnp.full((L,), g, jnp.int32),
                                       iota * S + ss])
        cnt = cnt + jnp.where(iv != 0, 1.0, 0.0)
      npad_v[pl.ds(g * L, L)] = float(S) - cnt
      recip_v[pl.ds(g * L, L)] = 1.0 / jnp.maximum(cnt, 1.0)
    w0cs = [plsc.load_gather(w0_v, [zero16, colvs[c]])
            for c in range(H // L)]

    def enc_u(u, _):
      uv = jnp.full((L,), u, jnp.int32)
      nv = plsc.load_gather(npad_v, [uv])
      rv = plsc.load_gather(recip_v, [uv])
      rowvs = [jnp.full((L,), u * S + ss, jnp.int32) for ss in range(S)]
      for c in range(H // L):
        acc = plsc.load_gather(base_a, [rowvs[0], colvs[c]])
        for ss in range(1, S):
          acc = acc + plsc.load_gather(base_a, [rowvs[ss], colvs[c]])
        plsc.store_scatter(enc_v, [uv, colvs[c]],
                           (acc - nv * w0cs[c]) * rv)
      return _

    lax.fori_loop(0, UPW, enc_u, 0)
    pltpu.sync_copy(enc_v, enc_sh.at[pl.ds(sid * UPW, UPW), :])

  cps0 = [pltpu.async_copy(embed_hbm.at[eidx0_v.at[r]],
                           base_a.at[pl.ds(r * 128, 128), :], sem_g0)
          for r in range(2)]
  cps0.append(pltpu.async_copy(buf_hbm.at[hkey0_v.at[0]], hyp0_v, sem_g0))
  cps1 = [pltpu.async_copy(embed_hbm.at[eidx1_v.at[r]],
                           base_b.at[pl.ds(r * 128, 128), :], sem_g1)
          for r in range(2)]
  cps1.append(pltpu.async_copy(buf_hbm.at[hkey1_v.at[0]], hyp1_v, sem_g1))

  with scope("p_keys"):
    for g in range(U // L):
      utb_g = utb_v[pl.ds(g * L, L)]
      slot = plsc.load_gather(hwpi_v, [utb_g])
      keys_v[pl.ds(g * L, L)] = slot * POOL_W + ui_v[pl.ds(g * L, L)]

  with scope("p_barrier"):
    plsc.subcore_barrier()

  NG = HB // L

  def compute(base_v, hyp_v, hkey_v, hpos_v, nh, cps):
    with scope("p_gwait"):
      for cp in cps:
        cp.wait()
    for b in range(CH // HB):

      @pl.when(b * HB < nh)
      def _batch():
        if b > 0:
          pltpu.async_copy(buf_hbm.at[hkey_v.at[b]], hyp_v, sem_o).wait()
        hks = [hkey_v[b, pl.ds(j * L, L)] for j in range(NG)]
        lanevs = [b * HB + j * L + iota < nh for j in range(NG)]
        nb = jnp.minimum(nh - b * HB, HB)

        def find_body(i, us):
          for d in range(8):
            p = i * 8 + d
            pk = plsc.load_gather(keys_v, [jnp.full((L,), p, jnp.int32)])
            us = tuple(jnp.where(pk == hks[j], p, us[j]) for j in range(NG))
          return us

        us = lax.fori_loop(0, U // 8, find_body, (neg1,) * NG)
        valids = [jnp.logical_and(us[j] >= 0, lanevs[j]) for j in range(NG)]
        anyv = jnp.int32(0)
        for j in range(NG):
          mapw_v[pl.ds(j * L, L)] = jnp.clip(us[j], 0, U - 1)
          uval_v[pl.ds(j * L, L)] = jnp.where(valids[j], 1, 0)
          anyv = anyv + jnp.sum(jnp.where(valids[j], 1, 0))

        def rp_tok(j, _):
          jv = jnp.full((L,), j, jnp.int32)
          posv = jnp.clip(plsc.load_gather(hpos_v, [b * HB + jv]), 0, CH - 1)
          for c in range(H // L):
            v = plsc.load_gather(hyp_v, [jv, colvs[c]])
            plsc.store_scatter(base_v, [posv, colvs[c]], v)
          return _

        lax.fori_loop(0, nb, rp_tok, 0)

        @pl.when(anyv > 0)
        def _override():
          pltpu.async_copy(enc_sh.at[mapw_v], srow_v, sem_o).wait()

          def ov_tok(j, _):
            jv = jnp.full((L,), j, jnp.int32)
            uvalid = plsc.load_gather(uval_v, [jv])

            @pl.when(jnp.any(uvalid > 0))
            def _ov_one():
              posv = jnp.clip(plsc.load_gather(hpos_v, [b * HB + jv]), 0,
                              CH - 1)
              for c in range(H // L):
                v = plsc.load_gather(srow_v, [jv, colvs[c]])
                plsc.store_scatter(base_v, [posv, colvs[c]], v)
            return _

          lax.fori_loop(0, nb, ov_tok, 0)

  with scope("p_chunk0"):
    compute(base_a, hyp0_v, hkey0_v, hpos0_v, nh0, cps0)
    w0 = pltpu.async_copy(base_a, out_hbm.at[pl.ds(tbase, CH), :], sem_w)
  with scope("p_chunk1"):
    compute(base_b, hyp1_v, hkey1_v, hpos1_v, nh1, cps1)
  with scope("p_tail"):
    w0.wait()
    pltpu.sync_copy(base_b, out_hbm.at[pl.ds(tbase + CH, CH), :])


@jax.jit
def _run(ids, embed, buf2d, upd_flat, ui, utb, hwpi, tb):
  mesh = plsc.VectorSubcoreMesh(core_axis_name="c", subcore_axis_name="s",
                                num_cores=NC, num_subcores=NS)
  scratch = [
      pltpu.VMEM((64,), jnp.int32),
      pltpu.VMEM((2, 128), jnp.int32),
      pltpu.VMEM((U,), jnp.int32),
      pltpu.VMEM((U,), jnp.int32),
      pltpu.VMEM((U,), jnp.int32),
      pltpu.VMEM((UPW, H), jnp.float32),
      pltpu.VMEM((CH,), jnp.int32),
      pltpu.VMEM((CH,), jnp.int32),
      pltpu.VMEM((CH,), jnp.int32),
      pltpu.VMEM((CH,), jnp.int32),
      pltpu.VMEM((2, 128), jnp.int32),
      pltpu.VMEM((2, 128), jnp.int32),
      pltpu.VMEM((4, HB), jnp.int32),
      pltpu.VMEM((4, HB), jnp.int32),
      pltpu.VMEM((CH,), jnp.int32),
      pltpu.VMEM((CH,), jnp.int32),
      pltpu.VMEM((HB,), jnp.int32),
      pltpu.VMEM((HB,), jnp.int32),
      pltpu.VMEM((UPW,), jnp.float32),
      pltpu.VMEM((UPW,), jnp.float32),
      pltpu.VMEM((1, H), jnp.float32),
      pltpu.VMEM((CH, H), jnp.float32),
      pltpu.VMEM((CH, H), jnp.float32),
      pltpu.VMEM((HB, H), jnp.float32),
      pltpu.VMEM((HB, H), jnp.float32),
      pltpu.VMEM((HB, H), jnp.float32),
      pltpu.VMEM_SHARED((U, H), jnp.float32),
      pltpu.SemaphoreType.DMA,
      pltpu.SemaphoreType.DMA,
      pltpu.SemaphoreType.DMA,
      pltpu.SemaphoreType.DMA,
      pltpu.SemaphoreType.DMA,
  ]
  f = pl.kernel(
      _body,
      out_type=jax.ShapeDtypeStruct((T, H), jnp.float32),
      mesh=mesh,
      scratch_types=scratch,
      compiler_params=pltpu.CompilerParams(needs_layout_passes=False),
  )
  return f(ids, embed, buf2d, upd_flat, ui, utb, hwpi, tb)


def kernel(input_, embed_weight, embedding_buffer, updates, updates_indices,
           update_to_batch, hyper_weight_pool_indices, token_to_batch_indices):
  ids = input_.astype(jnp.int32)
  buf2d = embedding_buffer.reshape(NKEY, H)
  upd_flat = updates.astype(jnp.int32).reshape(U * S)
  return _run(ids, embed_weight, buf2d, upd_flat,
              updates_indices.astype(jnp.int32),
              update_to_batch.astype(jnp.int32),
              hyper_weight_pool_indices.astype(jnp.int32),
              token_to_batch_indices.astype(jnp.int32))


# speedup vs baseline: 1.7565x; 1.1133x over previous
import jax
import jax.numpy as jnp
from jax import lax
from jax.experimental import pallas as pl
from jax.experimental.pallas import tpu as pltpu
from jax.experimental.pallas import tpu_sc as plsc

IVS = 100000
POOL_W = 2048
NKEY = 64 * POOL_W
T = 16384
H = 128
U = 512
S = 8
NC, NS, L = 2, 16, 16
NW = NC * NS
TPW = T // NW
CH = 256
HB = 64
UPW = U // NS


def _body(ids_hbm, embed_hbm, buf_hbm, upd_hbm, ui_hbm, utb_hbm, hwpi_hbm,
          tb_hbm, out_hbm,
          hwpi_v, uidx_v, utb_v, ui_v, keys_v, enc_v,
          ids0_v, ids1_v, tb0_v, tb1_v,
          eidx0_v, eidx1_v, hkey0_v, hkey1_v, hpos0_v, hpos1_v,
          mapw0_v, mapw1_v, uval0_v, uval1_v, npad_v, recip_v, w0_v,
          base_a, base_b, hyp0_v, hyp1_v, srow_v,
          enc_sh, sem_e, sem_g0, sem_g1, sem_w, sem_o):
  cid = lax.axis_index("c")
  sid = lax.axis_index("s")
  wid = sid * NC + cid
  iota = lax.iota(jnp.int32, L)
  neg1 = jnp.full((L,), -1, jnp.int32)
  tbase = wid * TPW

  scope = jax.named_scope
  with scope("p_stage"):
    ucps = [pltpu.async_copy(upd_hbm.at[pl.ds(sid * UPW * S + r * 128, 128)],
                             uidx_v.at[r], sem_e) for r in range(2)]
    stage_cps = [
        pltpu.async_copy(hwpi_hbm, hwpi_v, sem_o),
        pltpu.async_copy(utb_hbm, utb_v, sem_o),
        pltpu.async_copy(ui_hbm, ui_v, sem_o),
        pltpu.async_copy(ids_hbm.at[pl.ds(tbase, CH)], ids0_v, sem_o),
        pltpu.async_copy(ids_hbm.at[pl.ds(tbase + CH, CH)], ids1_v, sem_o),
        pltpu.async_copy(tb_hbm.at[pl.ds(tbase, CH)], tb0_v, sem_o),
        pltpu.async_copy(tb_hbm.at[pl.ds(tbase + CH, CH)], tb1_v, sem_o),
        pltpu.async_copy(embed_hbm.at[pl.ds(0, 1), :], w0_v, sem_o),
    ]
    for cp in ucps:
      cp.wait()
    enc_cps = [pltpu.async_copy(embed_hbm.at[uidx_v.at[r]],
                                base_b.at[pl.ds(r * 128, 128), :], sem_e)
               for r in range(2)]
    for cp in stage_cps:
      cp.wait()

  def pass1(ids_v, tbv_v, eidx_v, hkey_v, hpos_v):
    for r in range(4):
      for j in range(4):
        hkey_v[r, pl.ds(j * L, L)] = iota + r * HB + j * L
    nh = jnp.int32(0)
    for g in range(CH // L):
      ids = ids_v[pl.ds(g * L, L)]
      tb = tbv_v[pl.ds(g * L, L)]
      slot = plsc.load_gather(hwpi_v, [tb])
      ishyp = ids >= IVS
      key = slot * POOL_W + (ids - IVS)
      incl = plsc.cumsum(jnp.where(ishyp, 1, 0))
      rank = nh + incl - 1
      rclip = jnp.clip(rank, 0, CH - 1)
      plsc.store_scatter(hkey_v, [rclip >> 6, rclip & (HB - 1)], key,
                         mask=ishyp)
      plsc.store_scatter(hpos_v, [rclip], g * L + iota, mask=ishyp)
      eidx_v[g // 8, pl.ds((g % 8) * L, L)] = jnp.where(
          ishyp, g * L + iota, ids)
      nh = nh + jnp.sum(jnp.where(ishyp, 1, 0))
    return nh

  with scope("p_pass1"):
    nh0 = pass1(ids0_v, tb0_v, eidx0_v, hkey0_v, hpos0_v)
    nh1 = pass1(ids1_v, tb1_v, eidx1_v, hkey1_v, hpos1_v)

  cps0 = [pltpu.async_copy(embed_hbm.at[eidx0_v.at[r]],
                           base_a.at[pl.ds(r * 128, 128), :], sem_g0)
          for r in range(2)]
  cps0.append(pltpu.async_copy(buf_hbm.at[hkey0_v.at[0]], hyp0_v, sem_g0))

  with scope("p_encwait"):
    for cp in enc_cps:
      cp.wait()
  with scope("p_encode"):
    zero16 = jnp.zeros((L,), jnp.int32)
    colvs = [c * L + iota for c in range(H // L)]
    for g in range(UPW // L):
      cnt = jnp.zeros((L,), jnp.float32)
      for ss in range(S):
        iv = plsc.load_gather(uidx_v, [jnp.full((L,), g, jnp.int32),
                                       iota * S + ss])
        cnt = cnt + jnp.where(iv != 0, 1.0, 0.0)
      npad_v[pl.ds(g * L, L)] = float(S) - cnt
      recip_v[pl.ds(g * L, L)] = 1.0 / jnp.maximum(cnt, 1.0)
    w0cs = [plsc.load_gather(w0_v, [zero16, colvs[c]])
            for c in range(H // L)]

    def enc_u(u, _):
      uv = jnp.full((L,), u, jnp.int32)
      nv = plsc.load_gather(npad_v, [uv])
      rv = plsc.load_gather(recip_v, [uv])
      rowvs = [jnp.full((L,), u * S + ss, jnp.int32) for ss in range(S)]
      for c in range(H // L):
        acc = plsc.load_gather(base_b, [rowvs[0], colvs[c]])
        for ss in range(1, S):
          acc = acc + plsc.load_gather(base_b, [rowvs[ss], colvs[c]])
        plsc.store_scatter(enc_v, [uv, colvs[c]],
                           (acc - nv * w0cs[c]) * rv)
      return _

    lax.fori_loop(0, UPW, enc_u, 0)
    pltpu.sync_copy(enc_v, enc_sh.at[pl.ds(sid * UPW, UPW), :])

  cps1 = [pltpu.async_copy(embed_hbm.at[eidx1_v.at[r]],
                           base_b.at[pl.ds(r * 128, 128), :], sem_g1)
          for r in range(2)]
  cps1.append(pltpu.async_copy(buf_hbm.at[hkey1_v.at[0]], hyp1_v, sem_g1))

  with scope("p_keys"):
    for g in range(U // L):
      utb_g = utb_v[pl.ds(g * L, L)]
      slot = plsc.load_gather(hwpi_v, [utb_g])
      keys_v[pl.ds(g * L, L)] = slot * POOL_W + ui_v[pl.ds(g * L, L)]

  NG = HB // L

  def find_scan(groups):
    n = len(groups)

    def find_body(i, us):
      for d in range(8):
        p = i * 8 + d
        pk = plsc.load_gather(keys_v, [jnp.full((L,), p, jnp.int32)])
        us = tuple(jnp.where(pk == groups[j], p, us[j]) for j in range(n))
      return us

    return lax.fori_loop(0, U // 8, find_body, (neg1,) * n)

  def store_finds(mapw_v, uval_v, b, us, nh):
    for j in range(NG):
      lanev = b * HB + j * L + iota < nh
      valid = jnp.logical_and(us[j] >= 0, lanev)
      mapw_v[b, pl.ds(j * L, L)] = jnp.clip(us[j], 0, U - 1)
      uval_v[b, pl.ds(j * L, L)] = jnp.where(valid, 1, 0)

  def rp_toks(base_v, hyp_v, hpos_v, b, nh):
    nb = jnp.minimum(nh - b * HB, HB)

    def rp_tok(j, _):
      jv = jnp.full((L,), j, jnp.int32)
      posv = jnp.clip(plsc.load_gather(hpos_v, [b * HB + jv]), 0, CH - 1)
      for c in range(H // L):
        v = plsc.load_gather(hyp_v, [jv, colvs[c]])
        plsc.store_scatter(base_v, [posv, colvs[c]], v)
      return _

    lax.fori_loop(0, nb, rp_tok, 0)

  with scope("p_find0"):
    @pl.when(jnp.logical_or(nh0 > 0, nh1 > 0))
    def _find_b0():
      hks = ([hkey0_v[0, pl.ds(j * L, L)] for j in range(NG)] +
             [hkey1_v[0, pl.ds(j * L, L)] for j in range(NG)])
      us = find_scan(hks)
      store_finds(mapw0_v, uval0_v, 0, us[:NG], nh0)
      store_finds(mapw1_v, uval1_v, 0, us[NG:], nh1)

  def prebarrier_chunk(base_v, hyp_v, hkey_v, hpos_v, mapw_v, uval_v, nh,
                       cps):
    with scope("p_gwait"):
      for cp in cps:
        cp.wait()

    @pl.when(nh > 0)
    def _rp0():
      rp_toks(base_v, hyp_v, hpos_v, 0, nh)

    for b in range(1, CH // HB):
      @pl.when(b * HB < nh)
      def _batch():
        pltpu.async_copy(buf_hbm.at[hkey_v.at[b]], hyp_v, sem_o).wait()
        us = find_scan([hkey_v[b, pl.ds(j * L, L)] for j in range(NG)])
        store_finds(mapw_v, uval_v, b, us, nh)
        rp_toks(base_v, hyp_v, hpos_v, b, nh)

  with scope("p_chunk0"):
    prebarrier_chunk(base_a, hyp0_v, hkey0_v, hpos0_v, mapw0_v, uval0_v,
                     nh0, cps0)
  with scope("p_chunk1"):
    prebarrier_chunk(base_b, hyp1_v, hkey1_v, hpos1_v, mapw1_v, uval1_v,
                     nh1, cps1)

  with scope("p_barrier"):
    plsc.subcore_barrier()

  def overrides(base_v, hpos_v, mapw_v, uval_v, nh):
    for b in range(CH // HB):
      cnt = jnp.int32(0)
      for j in range(NG):
        cnt = cnt + jnp.sum(uval_v[b, pl.ds(j * L, L)])
      anyv = jnp.where(b * HB < nh, cnt, 0)

      @pl.when(anyv > 0)
      def _override():
        pltpu.async_copy(enc_sh.at[mapw_v.at[b]], srow_v, sem_o).wait()
        nb = jnp.minimum(nh - b * HB, HB)

        def ov_tok(j, _):
          jv = jnp.full((L,), j, jnp.int32)
          uvalid = plsc.load_gather(uval_v, [jnp.full((L,), b, jnp.int32),
                                             jv])

          @pl.when(jnp.any(uvalid > 0))
          def _ov_one():
            posv = jnp.clip(plsc.load_gather(hpos_v, [b * HB + jv]), 0,
                            CH - 1)
            for c in range(H // L):
              v = plsc.load_gather(srow_v, [jv, colvs[c]])
              plsc.store_scatter(base_v, [posv, colvs[c]], v)
          return _

        lax.fori_loop(0, nb, ov_tok, 0)

  with scope("p_post"):
    overrides(base_a, hpos0_v, mapw0_v, uval0_v, nh0)
    w0 = pltpu.async_copy(base_a, out_hbm.at[pl.ds(tbase, CH), :], sem_w)
    overrides(base_b, hpos1_v, mapw1_v, uval1_v, nh1)
  with scope("p_tail"):
    w0.wait()
    pltpu.sync_copy(base_b, out_hbm.at[pl.ds(tbase + CH, CH), :])


@jax.jit
def _run(ids, embed, buf2d, upd_flat, ui, utb, hwpi, tb):
  mesh = plsc.VectorSubcoreMesh(core_axis_name="c", subcore_axis_name="s",
                                num_cores=NC, num_subcores=NS)
  scratch = [
      pltpu.VMEM((64,), jnp.int32),
      pltpu.VMEM((2, 128), jnp.int32),
      pltpu.VMEM((U,), jnp.int32),
      pltpu.VMEM((U,), jnp.int32),
      pltpu.VMEM((U,), jnp.int32),
      pltpu.VMEM((UPW, H), jnp.float32),
      pltpu.VMEM((CH,), jnp.int32),
      pltpu.VMEM((CH,), jnp.int32),
      pltpu.VMEM((CH,), jnp.int32),
      pltpu.VMEM((CH,), jnp.int32),
      pltpu.VMEM((2, 128), jnp.int32),
      pltpu.VMEM((2, 128), jnp.int32),
      pltpu.VMEM((4, HB), jnp.int32),
      pltpu.VMEM((4, HB), jnp.int32),
      pltpu.VMEM((CH,), jnp.int32),
      pltpu.VMEM((CH,), jnp.int32),
      pltpu.VMEM((CH // HB, HB), jnp.int32),
      pltpu.VMEM((CH // HB, HB), jnp.int32),
      pltpu.VMEM((CH // HB, HB), jnp.int32),
      pltpu.VMEM((CH // HB, HB), jnp.int32),
      pltpu.VMEM((UPW,), jnp.float32),
      pltpu.VMEM((UPW,), jnp.float32),
      pltpu.VMEM((1, H), jnp.float32),
      pltpu.VMEM((CH, H), jnp.float32),
      pltpu.VMEM((CH, H), jnp.float32),
      pltpu.VMEM((HB, H), jnp.float32),
      pltpu.VMEM((HB, H), jnp.float32),
      pltpu.VMEM((HB, H), jnp.float32),
      pltpu.VMEM_SHARED((U, H), jnp.float32),
      pltpu.SemaphoreType.DMA,
      pltpu.SemaphoreType.DMA,
      pltpu.SemaphoreType.DMA,
      pltpu.SemaphoreType.DMA,
      pltpu.SemaphoreType.DMA,
  ]
  f = pl.kernel(
      _body,
      out_type=jax.ShapeDtypeStruct((T, H), jnp.float32),
      mesh=mesh,
      scratch_types=scratch,
      compiler_params=pltpu.CompilerParams(needs_layout_passes=False),
  )
  return f(ids, embed, buf2d, upd_flat, ui, utb, hwpi, tb)


def kernel(input_, embed_weight, embedding_buffer, updates, updates_indices,
           update_to_batch, hyper_weight_pool_indices, token_to_batch_indices):
  ids = input_.astype(jnp.int32)
  buf2d = embedding_buffer.reshape(NKEY, H)
  upd_flat = updates.astype(jnp.int32).reshape(U * S)
  return _run(ids, embed_weight, buf2d, upd_flat,
              updates_indices.astype(jnp.int32),
              update_to_batch.astype(jnp.int32),
              hyper_weight_pool_indices.astype(jnp.int32),
              token_to_batch_indices.astype(jnp.int32))
